# baseline (speedup 1.0000x reference)
import numpy as np
import jax
import jax.numpy as jnp
from jax.experimental import pallas as pl

B, H, W, DIM, WS, SHIFT, HEADS, DEPTH, MLPR = 4, 56, 56, 192, 7, 3, 6, 2, 4
HD = DIM // HEADS
NT = WS * WS
NWH = H // WS
ROWS = WS * W
GRID = B * NWH
SCALE = HD ** -0.5


def _rel_pos_index_np():
    ch, cw = np.meshgrid(np.arange(WS), np.arange(WS), indexing='ij')
    coords = np.stack([ch, cw]).reshape(2, -1)
    rel = coords[:, :, None] - coords[:, None, :]
    rel = rel.transpose(1, 2, 0).astype(np.int64)
    rel[:, :, 0] += WS - 1
    rel[:, :, 1] += WS - 1
    rel[:, :, 0] *= 2 * WS - 1
    return rel.sum(-1)


_REL_IDX = np.asarray(_rel_pos_index_np()).reshape(-1).astype(np.int32)


def _shift_mask_np():
    img = np.zeros((H, W), dtype=np.float32)
    cnt = 0
    for hs in (slice(0, -WS), slice(-WS, -SHIFT), slice(-SHIFT, None)):
        for ws_ in (slice(0, -WS), slice(-WS, -SHIFT), slice(-SHIFT, None)):
            img[hs, ws_] = cnt
            cnt += 1
    img = img.reshape(H // WS, WS, W // WS, WS).transpose(0, 2, 1, 3).reshape(-1, NT)
    diff = img[:, None, :] - img[:, :, None]
    return np.where(diff != 0, -100.0, 0.0).astype(np.float32)


_SHIFT_MASK = _shift_mask_np()


def _bias_kernel(idx_ref, rpb_ref, out_ref):
    idx = idx_ref[...]
    cols = jax.lax.broadcasted_iota(jnp.int32, (NT * NT, (2 * WS - 1) ** 2), 1)
    onehot = (idx == cols).astype(jnp.float32)
    out_ref[...] = jnp.dot(onehot, rpb_ref[...],
                           preferred_element_type=jnp.float32)


def _gather_bias(rpb):
    idx = jnp.asarray(_REL_IDX).reshape(NT * NT, 1)
    out = pl.pallas_call(
        _bias_kernel,
        out_shape=jax.ShapeDtypeStruct((NT * NT, HEADS), jnp.float32),
    )(idx, rpb)
    return out.reshape(NT, NT, HEADS).transpose(2, 0, 1)


def _to_windows(flat):
    c = flat.shape[-1]
    t = flat.reshape(WS, NWH, WS, c).transpose(1, 0, 2, 3)
    return t.reshape(NWH, NT, c)


def _from_windows(win):
    c = win.shape[-1]
    t = win.reshape(NWH, WS, WS, c).transpose(1, 0, 2, 3)
    return t.reshape(ROWS, c)


def _attn_block(X, mask3, qw_ref, qb_ref, pw_ref, pb_ref,
                n1w_ref, n1b_ref, n2w_ref, n2b_ref,
                f1w_ref, f1b_ref, f2w_ref, f2b_ref):
    mu = jnp.mean(X, axis=1, keepdims=True)
    xc = X - mu
    var = jnp.mean(xc * xc, axis=1, keepdims=True)
    xn = xc / jnp.sqrt(var + 1e-5) * n1w_ref[...] + n1b_ref[...]
    qkv = jnp.dot(xn, qw_ref[...],
                  preferred_element_type=jnp.float32) + qb_ref[...]
    qkvw = _to_windows(qkv)
    heads_out = []
    for h in range(HEADS):
        q = qkvw[:, :, HD * h:HD * (h + 1)]
        k = qkvw[:, :, DIM + HD * h:DIM + HD * (h + 1)]
        v = qkvw[:, :, 2 * DIM + HD * h:2 * DIM + HD * (h + 1)]
        s = jax.lax.dot_general(
            q, k, (((2,), (2,)), ((0,), (0,))),
            preferred_element_type=jnp.float32) * SCALE
        s = s + mask3(h)
        s = s - jnp.max(s, axis=2, keepdims=True)
        e = jnp.exp(s)
        p = e / jnp.sum(e, axis=2, keepdims=True)
        heads_out.append(jax.lax.dot_general(
            p, v, (((2,), (1,)), ((0,), (0,))),
            preferred_element_type=jnp.float32))
    y = _from_windows(jnp.concatenate(heads_out, axis=2))
    y = jnp.dot(y, pw_ref[...],
                preferred_element_type=jnp.float32) + pb_ref[...]
    x1 = X + y
    mu2 = jnp.mean(x1, axis=1, keepdims=True)
    xc2 = x1 - mu2
    var2 = jnp.mean(xc2 * xc2, axis=1, keepdims=True)
    xn2 = xc2 / jnp.sqrt(var2 + 1e-5) * n2w_ref[...] + n2b_ref[...]
    hmid = jax.nn.gelu(jnp.dot(xn2, f1w_ref[...],
                               preferred_element_type=jnp.float32)
                       + f1b_ref[...])
    return x1 + jnp.dot(hmid, f2w_ref[...],
                        preferred_element_type=jnp.float32) + f2b_ref[...]


def _block0_kernel(x_ref, mask_ref, *rest):
    o_ref, o2_ref = rest[-2], rest[-1]
    X = x_ref[0].reshape(ROWS, DIM)
    out = _attn_block(X, lambda h: mask_ref[0, h], *rest[:-2])
    out = out.reshape(WS, W, DIM)
    o_ref[0] = out
    o2_ref[0] = out


def _block1_kernel(xa_ref, xb_ref, mask_ref, *rest):
    o_ref, o2_ref = rest[-2], rest[-1]
    Sr = jnp.concatenate([xa_ref[0, SHIFT:], xb_ref[0, :SHIFT]], axis=0)
    Sc = jnp.concatenate([Sr[:, SHIFT:, :], Sr[:, :SHIFT, :]], axis=1)
    X = Sc.reshape(ROWS, DIM)
    start = (pl.program_id(0) % NWH) * NWH
    out = _attn_block(X, lambda h: mask_ref[pl.ds(start, NWH), h], *rest[:-2])
    out = out.reshape(WS, W, DIM)
    o_ref[0] = out
    o2_ref[0] = out


def _unshift_kernel(xa_ref, xb_ref, o_ref):
    Sr = jnp.concatenate([xa_ref[0, WS - SHIFT:], xb_ref[0, :WS - SHIFT]],
                         axis=0)
    o_ref[0] = jnp.concatenate([Sr[:, W - SHIFT:, :], Sr[:, :W - SHIFT, :]],
                               axis=1)


def _const_spec(shape):
    nd = len(shape)
    return pl.BlockSpec(shape, lambda i: (0,) * nd)


def _param_specs():
    return [
        _const_spec((DIM, 3 * DIM)),
        _const_spec((1, 3 * DIM)),
        _const_spec((DIM, DIM)),
        _const_spec((1, DIM)),
        _const_spec((1, DIM)),
        _const_spec((1, DIM)),
        _const_spec((1, DIM)),
        _const_spec((1, DIM)),
        _const_spec((DIM, MLPR * DIM)),
        _const_spec((1, MLPR * DIM)),
        _const_spec((MLPR * DIM, DIM)),
        _const_spec((1, DIM)),
    ]


def _param_args(p):
    return (p['qkv_w'], p['qkv_b'].reshape(1, -1),
            p['proj_w'], p['proj_b'].reshape(1, -1),
            p['norm1_w'].reshape(1, -1), p['norm1_b'].reshape(1, -1),
            p['norm2_w'].reshape(1, -1), p['norm2_b'].reshape(1, -1),
            p['fc1_w'], p['fc1_b'].reshape(1, -1),
            p['fc2_w'], p['fc2_b'].reshape(1, -1))


_SLAB = (1, WS, W, DIM)
_OUT4 = jax.ShapeDtypeStruct((B * NWH, WS, W, DIM), jnp.float32)


def _slab_spec(shift_blocks):
    if shift_blocks == 0:
        return pl.BlockSpec(_SLAB, lambda i: (i, 0, 0, 0))
    return pl.BlockSpec(
        _SLAB,
        lambda i: ((i // NWH) * NWH + (i % NWH + shift_blocks) % NWH, 0, 0, 0))


def _run_block0(x4, mask, p, interpret=False):
    return pl.pallas_call(
        _block0_kernel,
        grid=(GRID,),
        in_specs=[_slab_spec(0), _const_spec((1, HEADS, NT, NT))]
        + _param_specs(),
        out_specs=[pl.BlockSpec(_SLAB, lambda i: (i, 0, 0, 0)),
                   _slab_spec(-1)],
        out_shape=[_OUT4, _OUT4],
        interpret=interpret,
    )(x4, mask, *_param_args(p))


def _run_block1(x4, x4next, mask, p, interpret=False):
    return pl.pallas_call(
        _block1_kernel,
        grid=(GRID,),
        in_specs=[_slab_spec(0), _slab_spec(0),
                  _const_spec((NWH * NWH, HEADS, NT, NT))] + _param_specs(),
        out_specs=[pl.BlockSpec(_SLAB, lambda i: (i, 0, 0, 0)),
                   _slab_spec(1)],
        out_shape=[_OUT4, _OUT4],
        interpret=interpret,
    )(x4, x4next, mask, *_param_args(p))


def _run_unshift(xs4prev, xs4, interpret=False):
    return pl.pallas_call(
        _unshift_kernel,
        grid=(GRID,),
        in_specs=[_slab_spec(0), _slab_spec(0)],
        out_specs=pl.BlockSpec(_SLAB, lambda i: (i, 0, 0, 0)),
        out_shape=_OUT4,
        interpret=interpret,
    )(xs4prev, xs4)


def kernel(x, blk0_norm1_w, blk0_norm1_b, blk0_qkv_w, blk0_qkv_b,
           blk0_proj_w, blk0_proj_b, blk0_rpb, blk0_norm2_w, blk0_norm2_b,
           blk0_fc1_w, blk0_fc1_b, blk0_fc2_w, blk0_fc2_b,
           blk1_norm1_w, blk1_norm1_b, blk1_qkv_w, blk1_qkv_b,
           blk1_proj_w, blk1_proj_b, blk1_rpb, blk1_norm2_w, blk1_norm2_b,
           blk1_fc1_w, blk1_fc1_b, blk1_fc2_w, blk1_fc2_b):
    p0 = dict(qkv_w=blk0_qkv_w, qkv_b=blk0_qkv_b, proj_w=blk0_proj_w,
              proj_b=blk0_proj_b, norm1_w=blk0_norm1_w, norm1_b=blk0_norm1_b,
              norm2_w=blk0_norm2_w, norm2_b=blk0_norm2_b, fc1_w=blk0_fc1_w,
              fc1_b=blk0_fc1_b, fc2_w=blk0_fc2_w, fc2_b=blk0_fc2_b)
    p1 = dict(qkv_w=blk1_qkv_w, qkv_b=blk1_qkv_b, proj_w=blk1_proj_w,
              proj_b=blk1_proj_b, norm1_w=blk1_norm1_w, norm1_b=blk1_norm1_b,
              norm2_w=blk1_norm2_w, norm2_b=blk1_norm2_b, fc1_w=blk1_fc1_w,
              fc1_b=blk1_fc1_b, fc2_w=blk1_fc2_w, fc2_b=blk1_fc2_b)

    mask0 = _gather_bias(blk0_rpb)[None]
    bias1 = _gather_bias(blk1_rpb)
    mask1 = bias1[None] + jnp.asarray(_SHIFT_MASK)[:, None]

    x4 = x.reshape(B * NWH, WS, W, DIM)
    y4, y4next = _run_block0(x4, mask0, p0)
    ys, ysprev = _run_block1(y4, y4next, mask1, p1)
    out = _run_unshift(ysprev, ys)
    return out.reshape(B, H * W, DIM)


# speedup vs baseline: 1.2857x; 1.2857x over previous
import numpy as np
import jax
import jax.numpy as jnp
from jax.experimental import pallas as pl

B, H, W, DIM, WS, SHIFT, HEADS, DEPTH, MLPR = 4, 56, 56, 192, 7, 3, 6, 2, 4
HD = DIM // HEADS
NT = WS * WS
NWH = H // WS
ROWS = WS * W
GRID = B * NWH
SCALE = HD ** -0.5


def _rel_pos_index_np():
    ch, cw = np.meshgrid(np.arange(WS), np.arange(WS), indexing='ij')
    coords = np.stack([ch, cw]).reshape(2, -1)
    rel = coords[:, :, None] - coords[:, None, :]
    rel = rel.transpose(1, 2, 0).astype(np.int64)
    rel[:, :, 0] += WS - 1
    rel[:, :, 1] += WS - 1
    rel[:, :, 0] *= 2 * WS - 1
    return rel.sum(-1)


_REL_IDX = np.asarray(_rel_pos_index_np()).reshape(-1).astype(np.int32)


def _shift_mask_np():
    img = np.zeros((H, W), dtype=np.float32)
    cnt = 0
    for hs in (slice(0, -WS), slice(-WS, -SHIFT), slice(-SHIFT, None)):
        for ws_ in (slice(0, -WS), slice(-WS, -SHIFT), slice(-SHIFT, None)):
            img[hs, ws_] = cnt
            cnt += 1
    img = img.reshape(H // WS, WS, W // WS, WS).transpose(0, 2, 1, 3).reshape(-1, NT)
    diff = img[:, None, :] - img[:, :, None]
    return np.where(diff != 0, -100.0, 0.0).astype(np.float32)


_SHIFT_MASK = _shift_mask_np()


def _bias_kernel(idx_ref, rpbt_ref, out_ref):
    idx = idx_ref[...]
    cols = jax.lax.broadcasted_iota(jnp.int32, (NT * NT, (2 * WS - 1) ** 2), 1)
    onehot = (idx == cols).astype(jnp.float32)
    out_ref[...] = jax.lax.dot_general(
        onehot, rpbt_ref[...], (((1,), (1,)), ((), ())),
        preferred_element_type=jnp.float32)


def _gather_bias(rpb):
    idx = jnp.asarray(_REL_IDX).reshape(NT * NT, 1)
    out = pl.pallas_call(
        _bias_kernel,
        out_shape=jax.ShapeDtypeStruct((NT * NT, HEADS), jnp.float32),
    )(idx, rpb.T)
    return out.reshape(NT, NT, HEADS).transpose(2, 0, 1)


def _to_windows(flat):
    c = flat.shape[-1]
    t = flat.reshape(WS, NWH, WS, c).transpose(1, 0, 2, 3)
    return t.reshape(NWH, NT, c)


def _from_windows(win):
    c = win.shape[-1]
    t = win.reshape(NWH, WS, WS, c).transpose(1, 0, 2, 3)
    return t.reshape(ROWS, c)


def _attn_block(X, mask3, qw_ref, qb_ref, pw_ref, pb_ref,
                n1w_ref, n1b_ref, n2w_ref, n2b_ref,
                f1w_ref, f1b_ref, f2wt_ref, f2b_ref):
    mu = jnp.mean(X, axis=1, keepdims=True)
    xc = X - mu
    var = jnp.mean(xc * xc, axis=1, keepdims=True)
    xn = xc / jnp.sqrt(var + 1e-5) * n1w_ref[...] + n1b_ref[...]
    qkv = jnp.dot(xn, qw_ref[...],
                  preferred_element_type=jnp.float32) + qb_ref[...]
    qkvw = _to_windows(qkv)
    heads_out = []
    for h in range(HEADS):
        q = qkvw[:, :, HD * h:HD * (h + 1)]
        k = qkvw[:, :, DIM + HD * h:DIM + HD * (h + 1)]
        v = qkvw[:, :, 2 * DIM + HD * h:2 * DIM + HD * (h + 1)]
        s = jax.lax.dot_general(
            q, k, (((2,), (2,)), ((0,), (0,))),
            preferred_element_type=jnp.float32) * SCALE
        s = s + mask3(h)
        s = s - jnp.max(s, axis=2, keepdims=True)
        e = jnp.exp(s)
        p = e / jnp.sum(e, axis=2, keepdims=True)
        heads_out.append(jax.lax.dot_general(
            p, v, (((2,), (1,)), ((0,), (0,))),
            preferred_element_type=jnp.float32))
    y = _from_windows(jnp.concatenate(heads_out, axis=2))
    y = jnp.dot(y, pw_ref[...],
                preferred_element_type=jnp.float32) + pb_ref[...]
    x1 = X + y
    mu2 = jnp.mean(x1, axis=1, keepdims=True)
    xc2 = x1 - mu2
    var2 = jnp.mean(xc2 * xc2, axis=1, keepdims=True)
    xn2 = xc2 / jnp.sqrt(var2 + 1e-5) * n2w_ref[...] + n2b_ref[...]
    hmid = jax.nn.gelu(jnp.dot(xn2, f1w_ref[...],
                               preferred_element_type=jnp.float32)
                       + f1b_ref[...])
    return x1 + jax.lax.dot_general(
        hmid, f2wt_ref[...], (((1,), (1,)), ((), ())),
        preferred_element_type=jnp.float32) + f2b_ref[...]


def _block0_kernel(x_ref, mask_ref, *rest):
    o_ref, o2_ref = rest[-2], rest[-1]
    X = x_ref[0].reshape(ROWS, DIM)
    out = _attn_block(X, lambda h: mask_ref[0, h], *rest[:-2])
    out = out.reshape(WS, W, DIM)
    o_ref[0] = out
    o2_ref[0] = out


def _block1_kernel(xa_ref, xb_ref, mask_ref, *rest):
    o_ref = rest[-1]
    Sr = jnp.concatenate([xa_ref[0, SHIFT:], xb_ref[0, :SHIFT]], axis=0)
    Sc = jnp.concatenate([Sr[:, SHIFT:, :], Sr[:, :SHIFT, :]], axis=1)
    X = Sc.reshape(ROWS, DIM)
    start = (pl.program_id(0) % NWH) * NWH
    out = _attn_block(X, lambda h: mask_ref[pl.ds(start, NWH), h], *rest[:-1])
    o_ref[0] = out.reshape(WS, W, DIM)


def _tin_kernel(xt_ref, o_ref):
    X = jnp.transpose(xt_ref[0])
    o_ref[...] = X.reshape(NWH, WS, W, DIM)


def _unshift_tout_kernel(xs_ref, o_ref):
    S = xs_ref[...].reshape(H, W, DIM)
    Sr = jnp.concatenate([S[H - SHIFT:], S[:H - SHIFT]], axis=0)
    Sc = jnp.concatenate([Sr[:, W - SHIFT:, :], Sr[:, :W - SHIFT, :]], axis=1)
    o_ref[0] = jnp.transpose(Sc.reshape(H * W, DIM))


def _const_spec(shape):
    nd = len(shape)
    return pl.BlockSpec(shape, lambda i: (0,) * nd)


def _param_specs():
    return [
        _const_spec((DIM, 3 * DIM)),
        _const_spec((1, 3 * DIM)),
        _const_spec((DIM, DIM)),
        _const_spec((1, DIM)),
        _const_spec((1, DIM)),
        _const_spec((1, DIM)),
        _const_spec((1, DIM)),
        _const_spec((1, DIM)),
        _const_spec((DIM, MLPR * DIM)),
        _const_spec((1, MLPR * DIM)),
        _const_spec((DIM, MLPR * DIM)),
        _const_spec((1, DIM)),
    ]


def _param_args(p):
    return (p['qkv_w'], p['qkv_b'].reshape(1, -1),
            p['proj_w'], p['proj_b'].reshape(1, -1),
            p['norm1_w'].reshape(1, -1), p['norm1_b'].reshape(1, -1),
            p['norm2_w'].reshape(1, -1), p['norm2_b'].reshape(1, -1),
            p['fc1_w'], p['fc1_b'].reshape(1, -1),
            p['fc2_w'].T, p['fc2_b'].reshape(1, -1))


_SLAB = (1, WS, W, DIM)
_OUT4 = jax.ShapeDtypeStruct((B * NWH, WS, W, DIM), jnp.float32)


def _slab_spec(shift_blocks):
    if shift_blocks == 0:
        return pl.BlockSpec(_SLAB, lambda i: (i, 0, 0, 0))
    return pl.BlockSpec(
        _SLAB,
        lambda i: ((i // NWH) * NWH + (i % NWH + shift_blocks) % NWH, 0, 0, 0))


def _run_block0(x4, mask, p, interpret=False):
    return pl.pallas_call(
        _block0_kernel,
        grid=(GRID,),
        in_specs=[_slab_spec(0), _const_spec((1, HEADS, NT, NT))]
        + _param_specs(),
        out_specs=[pl.BlockSpec(_SLAB, lambda i: (i, 0, 0, 0)),
                   _slab_spec(-1)],
        out_shape=[_OUT4, _OUT4],
        interpret=interpret,
    )(x4, mask, *_param_args(p))


def _run_block1(x4, x4next, mask, p, interpret=False):
    return pl.pallas_call(
        _block1_kernel,
        grid=(GRID,),
        in_specs=[_slab_spec(0), _slab_spec(0),
                  _const_spec((NWH * NWH, HEADS, NT, NT))] + _param_specs(),
        out_specs=pl.BlockSpec(_SLAB, lambda i: (i, 0, 0, 0)),
        out_shape=_OUT4,
        interpret=interpret,
    )(x4, x4next, mask, *_param_args(p))


_IMGT = (1, DIM, H * W)


def _run_tin(xt, interpret=False):
    return pl.pallas_call(
        _tin_kernel,
        grid=(B,),
        in_specs=[pl.BlockSpec(_IMGT, lambda i: (i, 0, 0))],
        out_specs=pl.BlockSpec((NWH, WS, W, DIM), lambda i: (i, 0, 0, 0)),
        out_shape=_OUT4,
        interpret=interpret,
    )(xt)


def _run_unshift_tout(xs4, interpret=False):
    return pl.pallas_call(
        _unshift_tout_kernel,
        grid=(B,),
        in_specs=[pl.BlockSpec((NWH, WS, W, DIM), lambda i: (i, 0, 0, 0))],
        out_specs=pl.BlockSpec(_IMGT, lambda i: (i, 0, 0)),
        out_shape=jax.ShapeDtypeStruct((B, DIM, H * W), jnp.float32),
        interpret=interpret,
    )(xs4)


def kernel(x, blk0_norm1_w, blk0_norm1_b, blk0_qkv_w, blk0_qkv_b,
           blk0_proj_w, blk0_proj_b, blk0_rpb, blk0_norm2_w, blk0_norm2_b,
           blk0_fc1_w, blk0_fc1_b, blk0_fc2_w, blk0_fc2_b,
           blk1_norm1_w, blk1_norm1_b, blk1_qkv_w, blk1_qkv_b,
           blk1_proj_w, blk1_proj_b, blk1_rpb, blk1_norm2_w, blk1_norm2_b,
           blk1_fc1_w, blk1_fc1_b, blk1_fc2_w, blk1_fc2_b):
    p0 = dict(qkv_w=blk0_qkv_w, qkv_b=blk0_qkv_b, proj_w=blk0_proj_w,
              proj_b=blk0_proj_b, norm1_w=blk0_norm1_w, norm1_b=blk0_norm1_b,
              norm2_w=blk0_norm2_w, norm2_b=blk0_norm2_b, fc1_w=blk0_fc1_w,
              fc1_b=blk0_fc1_b, fc2_w=blk0_fc2_w, fc2_b=blk0_fc2_b)
    p1 = dict(qkv_w=blk1_qkv_w, qkv_b=blk1_qkv_b, proj_w=blk1_proj_w,
              proj_b=blk1_proj_b, norm1_w=blk1_norm1_w, norm1_b=blk1_norm1_b,
              norm2_w=blk1_norm2_w, norm2_b=blk1_norm2_b, fc1_w=blk1_fc1_w,
              fc1_b=blk1_fc1_b, fc2_w=blk1_fc2_w, fc2_b=blk1_fc2_b)

    mask0 = _gather_bias(blk0_rpb)[None]
    bias1 = _gather_bias(blk1_rpb)
    mask1 = bias1[None] + jnp.asarray(_SHIFT_MASK)[:, None]

    xt = x.transpose(0, 2, 1)
    x4 = _run_tin(xt)
    y4, y4next = _run_block0(x4, mask0, p0)
    ys = _run_block1(y4, y4next, mask1, p1)
    yt = _run_unshift_tout(ys)
    return yt.transpose(0, 2, 1)


# speedup vs baseline: 1.3339x; 1.0375x over previous
import numpy as np
import jax
import jax.numpy as jnp
from jax.experimental import pallas as pl

B, H, W, DIM, WS, SHIFT, HEADS, DEPTH, MLPR = 4, 56, 56, 192, 7, 3, 6, 2, 4
HD = DIM // HEADS
NT = WS * WS
NWH = H // WS
ROWS = WS * W
GRID = B * NWH
SCALE = HD ** -0.5


def _rel_pos_index_np():
    ch, cw = np.meshgrid(np.arange(WS), np.arange(WS), indexing='ij')
    coords = np.stack([ch, cw]).reshape(2, -1)
    rel = coords[:, :, None] - coords[:, None, :]
    rel = rel.transpose(1, 2, 0).astype(np.int64)
    rel[:, :, 0] += WS - 1
    rel[:, :, 1] += WS - 1
    rel[:, :, 0] *= 2 * WS - 1
    return rel.sum(-1)


_REL_IDX = np.asarray(_rel_pos_index_np()).reshape(-1).astype(np.int32)


def _shift_mask_np():
    img = np.zeros((H, W), dtype=np.float32)
    cnt = 0
    for hs in (slice(0, -WS), slice(-WS, -SHIFT), slice(-SHIFT, None)):
        for ws_ in (slice(0, -WS), slice(-WS, -SHIFT), slice(-SHIFT, None)):
            img[hs, ws_] = cnt
            cnt += 1
    img = img.reshape(H // WS, WS, W // WS, WS).transpose(0, 2, 1, 3).reshape(-1, NT)
    diff = img[:, None, :] - img[:, :, None]
    return np.where(diff != 0, -100.0, 0.0).astype(np.float32)


_SHIFT_MASK = _shift_mask_np()


def _bias_kernel(idx_ref, rpbt_ref, out_ref):
    idx = idx_ref[...]
    cols = jax.lax.broadcasted_iota(jnp.int32, (NT * NT, (2 * WS - 1) ** 2), 1)
    onehot = (idx == cols).astype(jnp.float32)
    out_ref[...] = jax.lax.dot_general(
        onehot, rpbt_ref[...], (((1,), (1,)), ((), ())),
        preferred_element_type=jnp.float32)


def _gather_bias(rpb):
    idx = jnp.asarray(_REL_IDX).reshape(NT * NT, 1)
    out = pl.pallas_call(
        _bias_kernel,
        out_shape=jax.ShapeDtypeStruct((NT * NT, HEADS), jnp.float32),
    )(idx, rpb.T)
    return out.reshape(NT, NT, HEADS).transpose(2, 0, 1)


def _to_windows(flat):
    c = flat.shape[-1]
    t = flat.reshape(WS, NWH, WS, c).transpose(1, 0, 2, 3)
    return t.reshape(NWH, NT, c)


def _from_windows(win):
    c = win.shape[-1]
    t = win.reshape(NWH, WS, WS, c).transpose(1, 0, 2, 3)
    return t.reshape(ROWS, c)


def _attn_block(X, mask3, qw_ref, qb_ref, pw_ref, pb_ref,
                n1w_ref, n1b_ref, n2w_ref, n2b_ref,
                f1w_ref, f1b_ref, f2wt_ref, f2b_ref):
    bf = jnp.bfloat16
    mu = jnp.mean(X, axis=1, keepdims=True)
    xc = X - mu
    var = jnp.mean(xc * xc, axis=1, keepdims=True)
    xn = xc / jnp.sqrt(var + 1e-5) * n1w_ref[...] + n1b_ref[...]
    qkv = jnp.dot(xn.astype(bf), qw_ref[...],
                  preferred_element_type=jnp.float32) + qb_ref[...]
    qkvw = _to_windows(qkv.astype(bf))
    heads_out = []
    for h in range(HEADS):
        q = qkvw[:, :, HD * h:HD * (h + 1)]
        k = qkvw[:, :, DIM + HD * h:DIM + HD * (h + 1)]
        v = qkvw[:, :, 2 * DIM + HD * h:2 * DIM + HD * (h + 1)]
        s = jax.lax.dot_general(
            q, k, (((2,), (2,)), ((0,), (0,))),
            preferred_element_type=jnp.float32) * SCALE
        s = s + mask3(h)
        s = s - jnp.max(s, axis=2, keepdims=True)
        e = jnp.exp(s)
        p = e / jnp.sum(e, axis=2, keepdims=True)
        heads_out.append(jax.lax.dot_general(
            p.astype(bf), v, (((2,), (1,)), ((0,), (0,))),
            preferred_element_type=jnp.float32))
    y = _from_windows(jnp.concatenate(heads_out, axis=2))
    y = jnp.dot(y.astype(bf), pw_ref[...],
                preferred_element_type=jnp.float32) + pb_ref[...]
    x1 = X + y
    mu2 = jnp.mean(x1, axis=1, keepdims=True)
    xc2 = x1 - mu2
    var2 = jnp.mean(xc2 * xc2, axis=1, keepdims=True)
    xn2 = xc2 / jnp.sqrt(var2 + 1e-5) * n2w_ref[...] + n2b_ref[...]
    hmid = jax.nn.gelu(jnp.dot(xn2.astype(bf), f1w_ref[...],
                               preferred_element_type=jnp.float32)
                       + f1b_ref[...])
    return x1 + jax.lax.dot_general(
        hmid.astype(bf), f2wt_ref[...], (((1,), (1,)), ((), ())),
        preferred_element_type=jnp.float32) + f2b_ref[...]


def _block0_kernel(x_ref, mask_ref, *rest):
    o_ref, o2_ref = rest[-2], rest[-1]
    X = x_ref[0].reshape(ROWS, DIM)
    out = _attn_block(X, lambda h: mask_ref[0, h], *rest[:-2])
    out = out.reshape(WS, W, DIM)
    o_ref[0] = out
    o2_ref[0] = out


def _block1_kernel(xa_ref, xb_ref, mask_ref, *rest):
    o_ref = rest[-1]
    Sr = jnp.concatenate([xa_ref[0, SHIFT:], xb_ref[0, :SHIFT]], axis=0)
    Sc = jnp.concatenate([Sr[:, SHIFT:, :], Sr[:, :SHIFT, :]], axis=1)
    X = Sc.reshape(ROWS, DIM)
    start = (pl.program_id(0) % NWH) * NWH
    out = _attn_block(X, lambda h: mask_ref[pl.ds(start, NWH), h], *rest[:-1])
    o_ref[0] = out.reshape(WS, W, DIM)


def _tin_kernel(xt_ref, o_ref):
    X = jnp.transpose(xt_ref[0])
    o_ref[...] = X.reshape(NWH, WS, W, DIM)


def _unshift_tout_kernel(xs_ref, o_ref):
    S = xs_ref[...].reshape(H, W, DIM)
    Sr = jnp.concatenate([S[H - SHIFT:], S[:H - SHIFT]], axis=0)
    Sc = jnp.concatenate([Sr[:, W - SHIFT:, :], Sr[:, :W - SHIFT, :]], axis=1)
    o_ref[0] = jnp.transpose(Sc.reshape(H * W, DIM))


def _const_spec(shape):
    nd = len(shape)
    return pl.BlockSpec(shape, lambda i: (0,) * nd)


def _param_specs():
    return [
        _const_spec((DIM, 3 * DIM)),
        _const_spec((1, 3 * DIM)),
        _const_spec((DIM, DIM)),
        _const_spec((1, DIM)),
        _const_spec((1, DIM)),
        _const_spec((1, DIM)),
        _const_spec((1, DIM)),
        _const_spec((1, DIM)),
        _const_spec((DIM, MLPR * DIM)),
        _const_spec((1, MLPR * DIM)),
        _const_spec((DIM, MLPR * DIM)),
        _const_spec((1, DIM)),
    ]


def _param_args(p):
    bf = jnp.bfloat16
    return (p['qkv_w'].astype(bf), p['qkv_b'].reshape(1, -1),
            p['proj_w'].astype(bf), p['proj_b'].reshape(1, -1),
            p['norm1_w'].reshape(1, -1), p['norm1_b'].reshape(1, -1),
            p['norm2_w'].reshape(1, -1), p['norm2_b'].reshape(1, -1),
            p['fc1_w'].astype(bf), p['fc1_b'].reshape(1, -1),
            p['fc2_w'].T.astype(bf), p['fc2_b'].reshape(1, -1))


_SLAB = (1, WS, W, DIM)
_OUT4 = jax.ShapeDtypeStruct((B * NWH, WS, W, DIM), jnp.float32)


def _slab_spec(shift_blocks):
    if shift_blocks == 0:
        return pl.BlockSpec(_SLAB, lambda i: (i, 0, 0, 0))
    return pl.BlockSpec(
        _SLAB,
        lambda i: ((i // NWH) * NWH + (i % NWH + shift_blocks) % NWH, 0, 0, 0))


def _run_block0(x4, mask, p, interpret=False):
    return pl.pallas_call(
        _block0_kernel,
        grid=(GRID,),
        in_specs=[_slab_spec(0), _const_spec((1, HEADS, NT, NT))]
        + _param_specs(),
        out_specs=[pl.BlockSpec(_SLAB, lambda i: (i, 0, 0, 0)),
                   _slab_spec(-1)],
        out_shape=[_OUT4, _OUT4],
        interpret=interpret,
    )(x4, mask, *_param_args(p))


def _run_block1(x4, x4next, mask, p, interpret=False):
    return pl.pallas_call(
        _block1_kernel,
        grid=(GRID,),
        in_specs=[_slab_spec(0), _slab_spec(0),
                  _const_spec((NWH * NWH, HEADS, NT, NT))] + _param_specs(),
        out_specs=pl.BlockSpec(_SLAB, lambda i: (i, 0, 0, 0)),
        out_shape=_OUT4,
        interpret=interpret,
    )(x4, x4next, mask, *_param_args(p))


_IMGT = (1, DIM, H * W)


def _run_tin(xt, interpret=False):
    return pl.pallas_call(
        _tin_kernel,
        grid=(B,),
        in_specs=[pl.BlockSpec(_IMGT, lambda i: (i, 0, 0))],
        out_specs=pl.BlockSpec((NWH, WS, W, DIM), lambda i: (i, 0, 0, 0)),
        out_shape=_OUT4,
        interpret=interpret,
    )(xt)


def _run_unshift_tout(xs4, interpret=False):
    return pl.pallas_call(
        _unshift_tout_kernel,
        grid=(B,),
        in_specs=[pl.BlockSpec((NWH, WS, W, DIM), lambda i: (i, 0, 0, 0))],
        out_specs=pl.BlockSpec(_IMGT, lambda i: (i, 0, 0)),
        out_shape=jax.ShapeDtypeStruct((B, DIM, H * W), jnp.float32),
        interpret=interpret,
    )(xs4)


def kernel(x, blk0_norm1_w, blk0_norm1_b, blk0_qkv_w, blk0_qkv_b,
           blk0_proj_w, blk0_proj_b, blk0_rpb, blk0_norm2_w, blk0_norm2_b,
           blk0_fc1_w, blk0_fc1_b, blk0_fc2_w, blk0_fc2_b,
           blk1_norm1_w, blk1_norm1_b, blk1_qkv_w, blk1_qkv_b,
           blk1_proj_w, blk1_proj_b, blk1_rpb, blk1_norm2_w, blk1_norm2_b,
           blk1_fc1_w, blk1_fc1_b, blk1_fc2_w, blk1_fc2_b):
    p0 = dict(qkv_w=blk0_qkv_w, qkv_b=blk0_qkv_b, proj_w=blk0_proj_w,
              proj_b=blk0_proj_b, norm1_w=blk0_norm1_w, norm1_b=blk0_norm1_b,
              norm2_w=blk0_norm2_w, norm2_b=blk0_norm2_b, fc1_w=blk0_fc1_w,
              fc1_b=blk0_fc1_b, fc2_w=blk0_fc2_w, fc2_b=blk0_fc2_b)
    p1 = dict(qkv_w=blk1_qkv_w, qkv_b=blk1_qkv_b, proj_w=blk1_proj_w,
              proj_b=blk1_proj_b, norm1_w=blk1_norm1_w, norm1_b=blk1_norm1_b,
              norm2_w=blk1_norm2_w, norm2_b=blk1_norm2_b, fc1_w=blk1_fc1_w,
              fc1_b=blk1_fc1_b, fc2_w=blk1_fc2_w, fc2_b=blk1_fc2_b)

    mask0 = _gather_bias(blk0_rpb)[None]
    bias1 = _gather_bias(blk1_rpb)
    mask1 = bias1[None] + jnp.asarray(_SHIFT_MASK)[:, None]

    xt = x.transpose(0, 2, 1)
    x4 = _run_tin(xt)
    y4, y4next = _run_block0(x4, mask0, p0)
    ys = _run_block1(y4, y4next, mask1, p1)
    yt = _run_unshift_tout(ys)
    return yt.transpose(0, 2, 1)


# speedup vs baseline: 1.4590x; 1.0938x over previous
import numpy as np
import jax
import jax.numpy as jnp
from jax.experimental import pallas as pl

B, H, W, DIM, WS, SHIFT, HEADS, DEPTH, MLPR = 4, 56, 56, 192, 7, 3, 6, 2, 4
HD = DIM // HEADS
NT = WS * WS
NWH = H // WS
ROWS = WS * W
GRID = B * NWH
SCALE = HD ** -0.5


def _rel_pos_index_np():
    ch, cw = np.meshgrid(np.arange(WS), np.arange(WS), indexing='ij')
    coords = np.stack([ch, cw]).reshape(2, -1)
    rel = coords[:, :, None] - coords[:, None, :]
    rel = rel.transpose(1, 2, 0).astype(np.int64)
    rel[:, :, 0] += WS - 1
    rel[:, :, 1] += WS - 1
    rel[:, :, 0] *= 2 * WS - 1
    return rel.sum(-1)


_REL_IDX = np.asarray(_rel_pos_index_np()).reshape(-1).astype(np.int32)


def _shift_mask_np():
    img = np.zeros((H, W), dtype=np.float32)
    cnt = 0
    for hs in (slice(0, -WS), slice(-WS, -SHIFT), slice(-SHIFT, None)):
        for ws_ in (slice(0, -WS), slice(-WS, -SHIFT), slice(-SHIFT, None)):
            img[hs, ws_] = cnt
            cnt += 1
    img = img.reshape(H // WS, WS, W // WS, WS).transpose(0, 2, 1, 3).reshape(-1, NT)
    diff = img[:, None, :] - img[:, :, None]
    return np.where(diff != 0, -100.0, 0.0).astype(np.float32)


_SHIFT_MASK = _shift_mask_np()


def _bias_kernel(idx_ref, rpbt_ref, out_ref):
    idx = idx_ref[...]
    cols = jax.lax.broadcasted_iota(jnp.int32, (NT * NT, (2 * WS - 1) ** 2), 1)
    onehot = (idx == cols).astype(jnp.float32)
    out_ref[...] = jax.lax.dot_general(
        onehot, rpbt_ref[...], (((1,), (1,)), ((), ())),
        preferred_element_type=jnp.float32)


def _gather_bias(rpb):
    idx = jnp.asarray(_REL_IDX).reshape(NT * NT, 1)
    out = pl.pallas_call(
        _bias_kernel,
        out_shape=jax.ShapeDtypeStruct((NT * NT, HEADS), jnp.float32),
    )(idx, rpb.T)
    return out.reshape(NT, NT, HEADS).transpose(2, 0, 1)


def _to_windows(flat):
    c = flat.shape[-1]
    t = flat.reshape(WS, NWH, WS, c).transpose(1, 0, 2, 3)
    return t.reshape(NWH, NT, c)


def _from_windows(win):
    c = win.shape[-1]
    t = win.reshape(NWH, WS, WS, c).transpose(1, 0, 2, 3)
    return t.reshape(ROWS, c)


def _attn_block(X, mask3, qw_ref, qb_ref, pw_ref, pb_ref,
                n1w_ref, n1b_ref, n2w_ref, n2b_ref,
                f1w_ref, f1b_ref, f2wt_ref, f2b_ref):
    bf = jnp.bfloat16
    mu = jnp.mean(X, axis=1, keepdims=True)
    xc = X - mu
    var = jnp.mean(xc * xc, axis=1, keepdims=True)
    xn = xc / jnp.sqrt(var + 1e-5) * n1w_ref[...] + n1b_ref[...]
    qkv = jnp.dot(xn.astype(bf), qw_ref[...],
                  preferred_element_type=jnp.float32) + qb_ref[...]
    qkvw = _to_windows(qkv.astype(bf))
    heads_out = []
    for h in range(HEADS):
        q = qkvw[:, :, HD * h:HD * (h + 1)]
        k = qkvw[:, :, DIM + HD * h:DIM + HD * (h + 1)]
        v = qkvw[:, :, 2 * DIM + HD * h:2 * DIM + HD * (h + 1)]
        s = jax.lax.dot_general(
            q, k, (((2,), (2,)), ((0,), (0,))),
            preferred_element_type=jnp.float32)
        e = jnp.exp(s + mask3(h))
        p = e * (1.0 / jnp.sum(e, axis=2, keepdims=True))
        heads_out.append(jax.lax.dot_general(
            p.astype(bf), v, (((2,), (1,)), ((0,), (0,))),
            preferred_element_type=jnp.float32))
    y = _from_windows(jnp.concatenate(heads_out, axis=2))
    y = jnp.dot(y.astype(bf), pw_ref[...],
                preferred_element_type=jnp.float32) + pb_ref[...]
    x1 = X + y
    mu2 = jnp.mean(x1, axis=1, keepdims=True)
    xc2 = x1 - mu2
    var2 = jnp.mean(xc2 * xc2, axis=1, keepdims=True)
    xn2 = xc2 / jnp.sqrt(var2 + 1e-5) * n2w_ref[...] + n2b_ref[...]
    hmid = jax.nn.gelu(jnp.dot(xn2.astype(bf), f1w_ref[...],
                               preferred_element_type=jnp.float32)
                       + f1b_ref[...])
    return x1 + jax.lax.dot_general(
        hmid.astype(bf), f2wt_ref[...], (((1,), (1,)), ((), ())),
        preferred_element_type=jnp.float32) + f2b_ref[...]


def _block0_kernel(x_ref, mask_ref, *rest):
    o_ref, o2_ref = rest[-2], rest[-1]
    X = x_ref[0].reshape(ROWS, DIM)
    out = _attn_block(X, lambda h: mask_ref[0, h], *rest[:-2])
    out = out.reshape(WS, W, DIM)
    o_ref[0] = out
    o2_ref[0] = out


def _block1_kernel(xa_ref, xb_ref, mask_ref, *rest):
    o_ref = rest[-1]
    Sr = jnp.concatenate([xa_ref[0, SHIFT:], xb_ref[0, :SHIFT]], axis=0)
    Sc = jnp.concatenate([Sr[:, SHIFT:, :], Sr[:, :SHIFT, :]], axis=1)
    X = Sc.reshape(ROWS, DIM)
    start = (pl.program_id(0) % NWH) * NWH
    out = _attn_block(X, lambda h: mask_ref[pl.ds(start, NWH), h], *rest[:-1])
    o_ref[0] = out.reshape(WS, W, DIM)


def _tin_kernel(xt_ref, o_ref):
    X = jnp.transpose(xt_ref[0])
    o_ref[...] = X.reshape(NWH, WS, W, DIM)


def _unshift_tout_kernel(xs_ref, o_ref):
    S = xs_ref[...].reshape(H, W, DIM)
    Sr = jnp.concatenate([S[H - SHIFT:], S[:H - SHIFT]], axis=0)
    Sc = jnp.concatenate([Sr[:, W - SHIFT:, :], Sr[:, :W - SHIFT, :]], axis=1)
    o_ref[0] = jnp.transpose(Sc.reshape(H * W, DIM))


def _const_spec(shape):
    nd = len(shape)
    return pl.BlockSpec(shape, lambda i: (0,) * nd)


def _param_specs():
    return [
        _const_spec((DIM, 3 * DIM)),
        _const_spec((1, 3 * DIM)),
        _const_spec((DIM, DIM)),
        _const_spec((1, DIM)),
        _const_spec((1, DIM)),
        _const_spec((1, DIM)),
        _const_spec((1, DIM)),
        _const_spec((1, DIM)),
        _const_spec((DIM, MLPR * DIM)),
        _const_spec((1, MLPR * DIM)),
        _const_spec((DIM, MLPR * DIM)),
        _const_spec((1, DIM)),
    ]


_QSCALE = np.concatenate([np.full(DIM, SCALE, np.float32),
                          np.ones(2 * DIM, np.float32)])


def _param_args(p):
    bf = jnp.bfloat16
    qscale = jnp.asarray(_QSCALE)
    return ((p['qkv_w'] * qscale).astype(bf),
            (p['qkv_b'] * qscale).reshape(1, -1),
            p['proj_w'].astype(bf), p['proj_b'].reshape(1, -1),
            p['norm1_w'].reshape(1, -1), p['norm1_b'].reshape(1, -1),
            p['norm2_w'].reshape(1, -1), p['norm2_b'].reshape(1, -1),
            p['fc1_w'].astype(bf), p['fc1_b'].reshape(1, -1),
            p['fc2_w'].T.astype(bf), p['fc2_b'].reshape(1, -1))


_SLAB = (1, WS, W, DIM)
_OUT4 = jax.ShapeDtypeStruct((B * NWH, WS, W, DIM), jnp.float32)


def _slab_spec(shift_blocks):
    if shift_blocks == 0:
        return pl.BlockSpec(_SLAB, lambda i: (i, 0, 0, 0))
    return pl.BlockSpec(
        _SLAB,
        lambda i: ((i // NWH) * NWH + (i % NWH + shift_blocks) % NWH, 0, 0, 0))


def _run_block0(x4, mask, p, interpret=False):
    return pl.pallas_call(
        _block0_kernel,
        grid=(GRID,),
        in_specs=[_slab_spec(0), _const_spec((1, HEADS, NT, NT))]
        + _param_specs(),
        out_specs=[pl.BlockSpec(_SLAB, lambda i: (i, 0, 0, 0)),
                   _slab_spec(-1)],
        out_shape=[_OUT4, _OUT4],
        interpret=interpret,
    )(x4, mask, *_param_args(p))


def _run_block1(x4, x4next, mask, p, interpret=False):
    return pl.pallas_call(
        _block1_kernel,
        grid=(GRID,),
        in_specs=[_slab_spec(0), _slab_spec(0),
                  _const_spec((NWH * NWH, HEADS, NT, NT))] + _param_specs(),
        out_specs=pl.BlockSpec(_SLAB, lambda i: (i, 0, 0, 0)),
        out_shape=_OUT4,
        interpret=interpret,
    )(x4, x4next, mask, *_param_args(p))


_IMGT = (1, DIM, H * W)


def _run_tin(xt, interpret=False):
    return pl.pallas_call(
        _tin_kernel,
        grid=(B,),
        in_specs=[pl.BlockSpec(_IMGT, lambda i: (i, 0, 0))],
        out_specs=pl.BlockSpec((NWH, WS, W, DIM), lambda i: (i, 0, 0, 0)),
        out_shape=_OUT4,
        interpret=interpret,
    )(xt)


def _run_unshift_tout(xs4, interpret=False):
    return pl.pallas_call(
        _unshift_tout_kernel,
        grid=(B,),
        in_specs=[pl.BlockSpec((NWH, WS, W, DIM), lambda i: (i, 0, 0, 0))],
        out_specs=pl.BlockSpec(_IMGT, lambda i: (i, 0, 0)),
        out_shape=jax.ShapeDtypeStruct((B, DIM, H * W), jnp.float32),
        interpret=interpret,
    )(xs4)


def kernel(x, blk0_norm1_w, blk0_norm1_b, blk0_qkv_w, blk0_qkv_b,
           blk0_proj_w, blk0_proj_b, blk0_rpb, blk0_norm2_w, blk0_norm2_b,
           blk0_fc1_w, blk0_fc1_b, blk0_fc2_w, blk0_fc2_b,
           blk1_norm1_w, blk1_norm1_b, blk1_qkv_w, blk1_qkv_b,
           blk1_proj_w, blk1_proj_b, blk1_rpb, blk1_norm2_w, blk1_norm2_b,
           blk1_fc1_w, blk1_fc1_b, blk1_fc2_w, blk1_fc2_b):
    p0 = dict(qkv_w=blk0_qkv_w, qkv_b=blk0_qkv_b, proj_w=blk0_proj_w,
              proj_b=blk0_proj_b, norm1_w=blk0_norm1_w, norm1_b=blk0_norm1_b,
              norm2_w=blk0_norm2_w, norm2_b=blk0_norm2_b, fc1_w=blk0_fc1_w,
              fc1_b=blk0_fc1_b, fc2_w=blk0_fc2_w, fc2_b=blk0_fc2_b)
    p1 = dict(qkv_w=blk1_qkv_w, qkv_b=blk1_qkv_b, proj_w=blk1_proj_w,
              proj_b=blk1_proj_b, norm1_w=blk1_norm1_w, norm1_b=blk1_norm1_b,
              norm2_w=blk1_norm2_w, norm2_b=blk1_norm2_b, fc1_w=blk1_fc1_w,
              fc1_b=blk1_fc1_b, fc2_w=blk1_fc2_w, fc2_b=blk1_fc2_b)

    mask0 = _gather_bias(blk0_rpb)[None]
    bias1 = _gather_bias(blk1_rpb)
    mask1 = bias1[None] + jnp.asarray(_SHIFT_MASK)[:, None]

    xt = x.transpose(0, 2, 1)
    x4 = _run_tin(xt)
    y4, y4next = _run_block0(x4, mask0, p0)
    ys = _run_block1(y4, y4next, mask1, p1)
    yt = _run_unshift_tout(ys)
    return yt.transpose(0, 2, 1)


# speedup vs baseline: 1.7413x; 1.1935x over previous
import numpy as np
import jax
import jax.numpy as jnp
from jax.experimental import pallas as pl

B, H, W, DIM, WS, SHIFT, HEADS, DEPTH, MLPR = 4, 56, 56, 192, 7, 3, 6, 2, 4
HD = DIM // HEADS
NT = WS * WS
NWH = H // WS
ROWS = WS * W
GRID = B * NWH
SCALE = HD ** -0.5


def _rel_pos_index_np():
    ch, cw = np.meshgrid(np.arange(WS), np.arange(WS), indexing='ij')
    coords = np.stack([ch, cw]).reshape(2, -1)
    rel = coords[:, :, None] - coords[:, None, :]
    rel = rel.transpose(1, 2, 0).astype(np.int64)
    rel[:, :, 0] += WS - 1
    rel[:, :, 1] += WS - 1
    rel[:, :, 0] *= 2 * WS - 1
    return rel.sum(-1)


_REL_IDX = np.asarray(_rel_pos_index_np()).reshape(-1).astype(np.int32)


def _shift_mask_np():
    img = np.zeros((H, W), dtype=np.float32)
    cnt = 0
    for hs in (slice(0, -WS), slice(-WS, -SHIFT), slice(-SHIFT, None)):
        for ws_ in (slice(0, -WS), slice(-WS, -SHIFT), slice(-SHIFT, None)):
            img[hs, ws_] = cnt
            cnt += 1
    img = img.reshape(H // WS, WS, W // WS, WS).transpose(0, 2, 1, 3).reshape(-1, NT)
    diff = img[:, None, :] - img[:, :, None]
    return np.where(diff != 0, -100.0, 0.0).astype(np.float32)


_SHIFT_MASK = _shift_mask_np()


def _bias_kernel(idx_ref, rpbt_ref, out_ref):
    idx = idx_ref[...]
    cols = jax.lax.broadcasted_iota(jnp.int32, (NT * NT, (2 * WS - 1) ** 2), 1)
    onehot = (idx == cols).astype(jnp.float32)
    out_ref[...] = jax.lax.dot_general(
        onehot, rpbt_ref[...], (((1,), (1,)), ((), ())),
        preferred_element_type=jnp.float32)


def _gather_bias(rpb):
    idx = jnp.asarray(_REL_IDX).reshape(NT * NT, 1)
    out = pl.pallas_call(
        _bias_kernel,
        out_shape=jax.ShapeDtypeStruct((NT * NT, HEADS), jnp.float32),
    )(idx, rpb.T)
    return out.reshape(NT, NT, HEADS).transpose(2, 0, 1)


def _to_windows(flat):
    c = flat.shape[-1]
    nwr = flat.shape[0] // ROWS
    t = flat.reshape(nwr, WS, NWH, WS, c).transpose(0, 2, 1, 3, 4)
    return t.reshape(nwr * NWH, NT, c)


def _from_windows(win):
    c = win.shape[-1]
    nwr = win.shape[0] // NWH
    t = win.reshape(nwr, NWH, WS, WS, c).transpose(0, 2, 1, 3, 4)
    return t.reshape(nwr * ROWS, c)


def _attn_block(X, mask3, qw_ref, qb_ref, pw_ref, pb_ref,
                n1w_ref, n1b_ref, n2w_ref, n2b_ref,
                f1w_ref, f1b_ref, f2wt_ref, f2b_ref):
    bf = jnp.bfloat16
    mu = jnp.mean(X, axis=1, keepdims=True)
    xc = X - mu
    var = jnp.mean(xc * xc, axis=1, keepdims=True)
    xn = xc / jnp.sqrt(var + 1e-5) * n1w_ref[...] + n1b_ref[...]
    qkv = jnp.dot(xn.astype(bf), qw_ref[...],
                  preferred_element_type=jnp.float32) + qb_ref[...]
    qkvw = _to_windows(qkv.astype(bf))
    heads_out = []
    for h in range(HEADS):
        q = qkvw[:, :, HD * h:HD * (h + 1)]
        k = qkvw[:, :, DIM + HD * h:DIM + HD * (h + 1)]
        v = qkvw[:, :, 2 * DIM + HD * h:2 * DIM + HD * (h + 1)]
        s = jax.lax.dot_general(
            q, k, (((2,), (2,)), ((0,), (0,))),
            preferred_element_type=jnp.float32)
        e = jnp.exp(s + mask3(h))
        p = e * (1.0 / jnp.sum(e, axis=2, keepdims=True))
        heads_out.append(jax.lax.dot_general(
            p.astype(bf), v, (((2,), (1,)), ((0,), (0,))),
            preferred_element_type=jnp.float32))
    y = _from_windows(jnp.concatenate(heads_out, axis=2))
    y = jnp.dot(y.astype(bf), pw_ref[...],
                preferred_element_type=jnp.float32) + pb_ref[...]
    x1 = X + y
    mu2 = jnp.mean(x1, axis=1, keepdims=True)
    xc2 = x1 - mu2
    var2 = jnp.mean(xc2 * xc2, axis=1, keepdims=True)
    xn2 = xc2 / jnp.sqrt(var2 + 1e-5) * n2w_ref[...] + n2b_ref[...]
    hmid = jax.nn.gelu(jnp.dot(xn2.astype(bf), f1w_ref[...],
                               preferred_element_type=jnp.float32)
                       + f1b_ref[...])
    return x1 + jax.lax.dot_general(
        hmid.astype(bf), f2wt_ref[...], (((1,), (1,)), ((), ())),
        preferred_element_type=jnp.float32) + f2b_ref[...]


def _block0_kernel(x_ref, mask_ref, *rest):
    o_ref = rest[-1]
    X = x_ref[0].reshape(ROWS, DIM)
    out = _attn_block(X, lambda h: mask_ref[0, h], *rest[:-1])
    o_ref[0] = out.reshape(WS, W, DIM)


def _block1_full_kernel(y_ref, mask_ref, *rest):
    o_ref = rest[-1]
    S = y_ref[...].reshape(H, W, DIM)
    Sr = jnp.concatenate([S[SHIFT:], S[:SHIFT]], axis=0)
    Sc = jnp.concatenate([Sr[:, SHIFT:, :], Sr[:, :SHIFT, :]], axis=1)
    X = Sc.reshape(H * W, DIM)
    out = _attn_block(X, lambda h: mask_ref[:, h], *rest[:-1])
    O = out.reshape(H, W, DIM)
    Or = jnp.concatenate([O[H - SHIFT:], O[:H - SHIFT]], axis=0)
    Oc = jnp.concatenate([Or[:, W - SHIFT:, :], Or[:, :W - SHIFT, :]], axis=1)
    o_ref[0] = jnp.transpose(Oc.reshape(H * W, DIM))


def _tin_kernel(xt_ref, o_ref):
    X = jnp.transpose(xt_ref[0])
    o_ref[...] = X.reshape(NWH, WS, W, DIM)


def _const_spec(shape):
    nd = len(shape)
    return pl.BlockSpec(shape, lambda i: (0,) * nd)


def _param_specs():
    return [
        _const_spec((DIM, 3 * DIM)),
        _const_spec((1, 3 * DIM)),
        _const_spec((DIM, DIM)),
        _const_spec((1, DIM)),
        _const_spec((1, DIM)),
        _const_spec((1, DIM)),
        _const_spec((1, DIM)),
        _const_spec((1, DIM)),
        _const_spec((DIM, MLPR * DIM)),
        _const_spec((1, MLPR * DIM)),
        _const_spec((DIM, MLPR * DIM)),
        _const_spec((1, DIM)),
    ]


_QSCALE = np.concatenate([np.full(DIM, SCALE, np.float32),
                          np.ones(2 * DIM, np.float32)])


def _param_args(p):
    bf = jnp.bfloat16
    qscale = jnp.asarray(_QSCALE)
    return ((p['qkv_w'] * qscale).astype(bf),
            (p['qkv_b'] * qscale).reshape(1, -1),
            p['proj_w'].astype(bf), p['proj_b'].reshape(1, -1),
            p['norm1_w'].reshape(1, -1), p['norm1_b'].reshape(1, -1),
            p['norm2_w'].reshape(1, -1), p['norm2_b'].reshape(1, -1),
            p['fc1_w'].astype(bf), p['fc1_b'].reshape(1, -1),
            p['fc2_w'].T.astype(bf), p['fc2_b'].reshape(1, -1))


_SLAB = (1, WS, W, DIM)
_OUT4 = jax.ShapeDtypeStruct((B * NWH, WS, W, DIM), jnp.float32)


def _slab_spec(shift_blocks):
    if shift_blocks == 0:
        return pl.BlockSpec(_SLAB, lambda i: (i, 0, 0, 0))
    return pl.BlockSpec(
        _SLAB,
        lambda i: ((i // NWH) * NWH + (i % NWH + shift_blocks) % NWH, 0, 0, 0))


def _run_block0(x4, mask, p, interpret=False):
    return pl.pallas_call(
        _block0_kernel,
        grid=(GRID,),
        in_specs=[_slab_spec(0), _const_spec((1, HEADS, NT, NT))]
        + _param_specs(),
        out_specs=pl.BlockSpec(_SLAB, lambda i: (i, 0, 0, 0)),
        out_shape=_OUT4,
        interpret=interpret,
    )(x4, mask, *_param_args(p))


_IMGT = (1, DIM, H * W)


def _run_block1_full(y4, mask, p, interpret=False):
    return pl.pallas_call(
        _block1_full_kernel,
        grid=(B,),
        in_specs=[pl.BlockSpec((NWH, WS, W, DIM), lambda i: (i, 0, 0, 0)),
                  _const_spec((NWH * NWH, HEADS, NT, NT))] + _param_specs(),
        out_specs=pl.BlockSpec(_IMGT, lambda i: (i, 0, 0)),
        out_shape=jax.ShapeDtypeStruct((B, DIM, H * W), jnp.float32),
        interpret=interpret,
    )(y4, mask, *_param_args(p))


def _run_tin(xt, interpret=False):
    return pl.pallas_call(
        _tin_kernel,
        grid=(B,),
        in_specs=[pl.BlockSpec(_IMGT, lambda i: (i, 0, 0))],
        out_specs=pl.BlockSpec((NWH, WS, W, DIM), lambda i: (i, 0, 0, 0)),
        out_shape=_OUT4,
        interpret=interpret,
    )(xt)


def kernel(x, blk0_norm1_w, blk0_norm1_b, blk0_qkv_w, blk0_qkv_b,
           blk0_proj_w, blk0_proj_b, blk0_rpb, blk0_norm2_w, blk0_norm2_b,
           blk0_fc1_w, blk0_fc1_b, blk0_fc2_w, blk0_fc2_b,
           blk1_norm1_w, blk1_norm1_b, blk1_qkv_w, blk1_qkv_b,
           blk1_proj_w, blk1_proj_b, blk1_rpb, blk1_norm2_w, blk1_norm2_b,
           blk1_fc1_w, blk1_fc1_b, blk1_fc2_w, blk1_fc2_b):
    p0 = dict(qkv_w=blk0_qkv_w, qkv_b=blk0_qkv_b, proj_w=blk0_proj_w,
              proj_b=blk0_proj_b, norm1_w=blk0_norm1_w, norm1_b=blk0_norm1_b,
              norm2_w=blk0_norm2_w, norm2_b=blk0_norm2_b, fc1_w=blk0_fc1_w,
              fc1_b=blk0_fc1_b, fc2_w=blk0_fc2_w, fc2_b=blk0_fc2_b)
    p1 = dict(qkv_w=blk1_qkv_w, qkv_b=blk1_qkv_b, proj_w=blk1_proj_w,
              proj_b=blk1_proj_b, norm1_w=blk1_norm1_w, norm1_b=blk1_norm1_b,
              norm2_w=blk1_norm2_w, norm2_b=blk1_norm2_b, fc1_w=blk1_fc1_w,
              fc1_b=blk1_fc1_b, fc2_w=blk1_fc2_w, fc2_b=blk1_fc2_b)

    mask0 = _gather_bias(blk0_rpb)[None]
    bias1 = _gather_bias(blk1_rpb)
    mask1 = bias1[None] + jnp.asarray(_SHIFT_MASK)[:, None]

    xt = x.transpose(0, 2, 1)
    x4 = _run_tin(xt)
    y4 = _run_block0(x4, mask0, p0)
    yt = _run_block1_full(y4, mask1, p1)
    return yt.transpose(0, 2, 1)
